# baseline (speedup 1.0000x reference)
import jax
import jax.numpy as jnp
from jax.experimental import pallas as pl
from jax.experimental.pallas import tpu as pltpu

_ROWS = 100000
_CHUNK = 2000
_NCHUNKS = _ROWS // _CHUNK
_NSEM = 8


def _copy3_kernel(c_in, n_in, u_in, c_out, n_out, u_out, sems):
    copies = []
    for t, (src, dst) in enumerate(((c_in, c_out), (n_in, n_out), (u_in, u_out))):
        for i in range(_NCHUNKS):
            copies.append(pltpu.make_async_copy(
                src.at[pl.ds(i * _CHUNK, _CHUNK), :],
                dst.at[pl.ds(i * _CHUNK, _CHUNK), :],
                sems.at[(t * _NCHUNKS + i) % _NSEM]))
    for c in copies:
        c.start()
    for c in copies:
        c.wait()


def kernel(c_embeddings, n_embeddings, u_embeddings):
    out = pl.pallas_call(
        _copy3_kernel,
        in_specs=[pl.BlockSpec(memory_space=pl.ANY)] * 3,
        out_specs=[pl.BlockSpec(memory_space=pl.ANY)] * 3,
        out_shape=(
            jax.ShapeDtypeStruct(c_embeddings.shape, c_embeddings.dtype),
            jax.ShapeDtypeStruct(n_embeddings.shape, n_embeddings.dtype),
            jax.ShapeDtypeStruct(u_embeddings.shape, u_embeddings.dtype),
        ),
        scratch_shapes=[pltpu.SemaphoreType.DMA((_NSEM,))],
    )(c_embeddings, n_embeddings, u_embeddings)
    return (out[0], out[1], out[2])


# speedup vs baseline: 24.1116x; 24.1116x over previous
import jax
import jax.numpy as jnp
from jax import lax
from jax.experimental import pallas as pl
from jax.experimental.pallas import tpu as pltpu
from jax.experimental.pallas import tpu_sc as plsc

_ROWS = 100000
_NW = 32
_CHUNK = 200
_NCH = _ROWS // _CHUNK
_ITERS = -(-_NCH // _NW)
_NBUF = 2


def _in_cp(src, cid, bufs, sems, i):
    b = i % _NBUF
    return pltpu.make_async_copy(
        src.at[pl.ds(cid * _CHUNK, _CHUNK), :], bufs.at[b], sems.at[b])


def _out_cp(dst, cid, bufs, sems, k):
    b = k % _NBUF
    return pltpu.make_async_copy(
        bufs.at[b], dst.at[pl.ds(cid * _CHUNK, _CHUNK), :], sems.at[b])


def _pipe(src, dst, wid, bufs, in_sems, out_sems):
    for i in range(_ITERS + _NBUF):
        kd = i - _NBUF
        if 0 <= kd < _ITERS:
            cd = wid + kd * _NW

            @pl.when(cd < _NCH)
            def _(cd=cd, kd=kd):
                _out_cp(dst, cd, bufs, out_sems, kd).wait()
        if i < _ITERS:
            ci = wid + i * _NW

            @pl.when(ci < _NCH)
            def _(ci=ci, i=i):
                _in_cp(src, ci, bufs, in_sems, i).start()
        k = i - 1
        if 0 <= k < _ITERS:
            ck = wid + k * _NW

            @pl.when(ck < _NCH)
            def _(ck=ck, k=k):
                _in_cp(src, ck, bufs, in_sems, k).wait()
                _out_cp(dst, ck, bufs, out_sems, k).start()


def _sc_body(c_in, n_in, u_in, c_out, n_out, u_out,
             buf128, buf64, in_sems, out_sems):
    wid = lax.axis_index("s") * 2 + lax.axis_index("c")
    _pipe(c_in, c_out, wid, buf128, in_sems, out_sems)
    _pipe(n_in, n_out, wid, buf128, in_sems, out_sems)
    _pipe(u_in, u_out, wid, buf64, in_sems, out_sems)


def kernel(c_embeddings, n_embeddings, u_embeddings):
    mesh = plsc.VectorSubcoreMesh(
        core_axis_name="c", subcore_axis_name="s", num_cores=2, num_subcores=16)
    run = pl.kernel(
        _sc_body,
        out_type=(
            jax.ShapeDtypeStruct(c_embeddings.shape, c_embeddings.dtype),
            jax.ShapeDtypeStruct(n_embeddings.shape, n_embeddings.dtype),
            jax.ShapeDtypeStruct(u_embeddings.shape, u_embeddings.dtype),
        ),
        mesh=mesh,
        scratch_types=[
            pltpu.VMEM((_NBUF, _CHUNK, 128), jnp.float32),
            pltpu.VMEM((_NBUF, _CHUNK, 64), jnp.float32),
            pltpu.SemaphoreType.DMA((_NBUF,)),
            pltpu.SemaphoreType.DMA((_NBUF,)),
        ],
    )
    out = run(c_embeddings, n_embeddings, u_embeddings)
    return (out[0], out[1], out[2])
